# Initial kernel scaffold; baseline (speedup 1.0000x reference)
import jax
import jax.numpy as jnp
from jax.experimental import pallas as pl


def kernel(x, W_qkv, b_qkv, W_o, b_o):
    raise NotImplementedError("write your pallas kernel here")


# speedup vs baseline: 1.0325x; 1.0325x over previous
import jax
import jax.numpy as jnp
from jax.experimental import pallas as pl
from jax.experimental.pallas import tpu as pltpu

N = 2
P2 = 64
W2 = 64
DIM = 1024
QK = 1024
KV = 2048
TOPK = 4
SCALE = QK ** -0.5
PB = 8


def _dot(a, b):
    return jax.lax.dot_general(a, b, (((1,), (0,)), ((), ())),
                               preferred_element_type=jnp.float32)


def _dot_t(a, b):
    return jax.lax.dot_general(a, b, (((1,), (1,)), ((), ())),
                               preferred_element_type=jnp.float32)


def _qkv_kernel(x_ref, w_ref, b_ref, q_ref, kv_ref, qw_ref, kw_ref):
    x = x_ref[...]
    qkv = _dot(x, w_ref[...]) + b_ref[...]
    q = qkv[:, :QK]
    kv = qkv[:, QK:]
    q_ref[...] = q
    kv_ref[...] = kv
    r = jax.lax.broadcasted_iota(jnp.int32, (PB, PB * W2), 0)
    c = jax.lax.broadcasted_iota(jnp.int32, (PB, PB * W2), 1) // W2
    pool = jnp.where(r == c, 1.0 / W2, 0.0)
    qw_ref[...] = _dot(pool, q)
    kw_ref[...] = _dot(pool, kv[:, :QK])


def _route_kernel(qw_ref, kw_ref, idx_ref, wgt_ref):
    qw = qw_ref[...]
    kw = kw_ref[...]
    logit = _dot_t(qw * SCALE, kw)
    col = jax.lax.broadcasted_iota(jnp.int32, (P2, P2), 1)
    lane = jax.lax.broadcasted_iota(jnp.int32, (P2, 128), 1)
    idx_out = jnp.zeros((P2, 128), jnp.int32)
    val_out = jnp.zeros((P2, 128), jnp.float32)
    cur = logit
    for t in range(TOPK):
        m = jnp.max(cur, axis=-1, keepdims=True)
        a = jnp.min(jnp.where(cur == m, col, P2), axis=-1, keepdims=True)
        idx_out = jnp.where(lane == t, a, idx_out)
        val_out = jnp.where(lane == t, m, val_out)
        cur = jnp.where(col == a, -jnp.inf, cur)
    e = jnp.where(lane < TOPK, jnp.exp(val_out - val_out[:, :1]), 0.0)
    s = jnp.sum(e, axis=-1, keepdims=True)
    idx_ref[...] = idx_out
    wgt_ref[...] = e / s


def _attn_kernel(ridx_ref, q_ref, kv_ref, rw_ref, wo_ref, bo_ref, o_ref,
                 acc_ref, m_ref, s_ref):
    t = pl.program_id(1)

    @pl.when(t == 0)
    def _():
        m_ref[...] = jnp.full((W2, 128), -jnp.inf, jnp.float32)
        s_ref[...] = jnp.zeros((W2, 128), jnp.float32)
        acc_ref[...] = jnp.zeros((W2, DIM), jnp.float32)

    q = q_ref[...]
    kvt = kv_ref[...]
    k = kvt[:, :QK]
    v = kvt[:, QK:]
    lane = jax.lax.broadcasted_iota(jnp.int32, (1, 128), 1)
    w = jnp.sum(jnp.where(lane == t, rw_ref[0], 0.0))

    l = _dot_t(q * SCALE, k) * w
    m_prev = m_ref[:, :1]
    m_new = jnp.maximum(m_prev, jnp.max(l, axis=-1, keepdims=True))
    alpha = jnp.exp(m_prev - m_new)
    p = jnp.exp(l - m_new)
    s_ref[...] = jnp.broadcast_to(s_ref[:, :1] * alpha
                                  + jnp.sum(p, axis=-1, keepdims=True),
                                  (W2, 128))
    m_ref[...] = jnp.broadcast_to(m_new, (W2, 128))
    acc_ref[...] = acc_ref[...] * alpha + _dot(p * w, v)

    @pl.when(t == TOPK - 1)
    def _():
        o = acc_ref[...] / s_ref[:, :1]
        o_ref[...] = _dot(o, wo_ref[...]) + bo_ref[...]


def kernel(x, W_qkv, b_qkv, W_o, b_o):
    n, p2, w2, dim = x.shape
    rows = n * p2 * w2
    x2 = x.reshape(rows, dim)
    b2 = b_qkv.reshape(1, 2 * QK + DIM)

    q2, kv2, qw, kw = pl.pallas_call(
        _qkv_kernel,
        grid=(rows // (PB * W2),),
        in_specs=[
            pl.BlockSpec((PB * W2, DIM), lambda g: (g, 0)),
            pl.BlockSpec((DIM, 2 * QK + DIM), lambda g: (0, 0)),
            pl.BlockSpec((1, 2 * QK + DIM), lambda g: (0, 0)),
        ],
        out_specs=[
            pl.BlockSpec((PB * W2, QK), lambda g: (g, 0)),
            pl.BlockSpec((PB * W2, KV), lambda g: (g, 0)),
            pl.BlockSpec((PB, QK), lambda g: (g, 0)),
            pl.BlockSpec((PB, QK), lambda g: (g, 0)),
        ],
        out_shape=[
            jax.ShapeDtypeStruct((rows, QK), jnp.float32),
            jax.ShapeDtypeStruct((rows, KV), jnp.float32),
            jax.ShapeDtypeStruct((n * p2, QK), jnp.float32),
            jax.ShapeDtypeStruct((n * p2, QK), jnp.float32),
        ],
    )(x2, W_qkv, b2)

    r_idx, r_wgt = pl.pallas_call(
        _route_kernel,
        grid=(n,),
        in_specs=[
            pl.BlockSpec((P2, QK), lambda b: (b, 0)),
            pl.BlockSpec((P2, QK), lambda b: (b, 0)),
        ],
        out_specs=[
            pl.BlockSpec((P2, 128), lambda b: (b, 0)),
            pl.BlockSpec((P2, 128), lambda b: (b, 0)),
        ],
        out_shape=[
            jax.ShapeDtypeStruct((n * p2, 128), jnp.int32),
            jax.ShapeDtypeStruct((n * p2, 128), jnp.float32),
        ],
    )(qw, kw)

    rw3 = r_wgt.reshape(n * p2, 1, 128)
    bo2 = b_o.reshape(1, DIM)

    out = pl.pallas_call(
        _attn_kernel,
        grid_spec=pltpu.PrefetchScalarGridSpec(
            num_scalar_prefetch=1,
            grid=(n * p2, TOPK),
            in_specs=[
                pl.BlockSpec((W2, QK), lambda i, t, ridx: (i, 0)),
                pl.BlockSpec(
                    (W2, KV),
                    lambda i, t, ridx: ((i // P2) * P2 + ridx[i, t], 0)),
                pl.BlockSpec((1, 1, 128), lambda i, t, ridx: (i, 0, 0)),
                pl.BlockSpec((DIM, DIM), lambda i, t, ridx: (0, 0)),
                pl.BlockSpec((1, DIM), lambda i, t, ridx: (0, 0)),
            ],
            out_specs=pl.BlockSpec((W2, DIM), lambda i, t, ridx: (i, 0)),
            scratch_shapes=[
                pltpu.VMEM((W2, DIM), jnp.float32),
                pltpu.VMEM((W2, 128), jnp.float32),
                pltpu.VMEM((W2, 128), jnp.float32),
            ],
        ),
        out_shape=jax.ShapeDtypeStruct((rows, DIM), jnp.float32),
        compiler_params=pltpu.CompilerParams(
            dimension_semantics=("arbitrary", "arbitrary")),
    )(r_idx, q2, kv2, rw3, W_o, bo2)

    return out.reshape(n, p2, w2, dim)


# speedup vs baseline: 2.1317x; 2.0646x over previous
import jax
import jax.numpy as jnp
from jax.experimental import pallas as pl
from jax.experimental.pallas import tpu as pltpu

N = 2
P2 = 64
W2 = 64
DIM = 1024
QK = 1024
KV = 2048
TOPK = 4
SCALE = QK ** -0.5
PB = 8


def _dot(a, b):
    return jax.lax.dot_general(a, b, (((1,), (0,)), ((), ())),
                               preferred_element_type=jnp.float32)


def _dot_t(a, b):
    return jax.lax.dot_general(a, b, (((1,), (1,)), ((), ())),
                               preferred_element_type=jnp.float32)


def _qkv_kernel(x_ref, w_ref, b_ref, q_ref, kv_ref, qw_ref, kw_ref):
    x = x_ref[...]
    qkv = _dot(x, w_ref[...]) + b_ref[...]
    q = qkv[:, :QK]
    kv = qkv[:, QK:]
    q_ref[...] = q
    kv_ref[...] = kv
    r = jax.lax.broadcasted_iota(jnp.int32, (PB, PB * W2), 0)
    c = jax.lax.broadcasted_iota(jnp.int32, (PB, PB * W2), 1) // W2
    pool = jnp.where(r == c, 1.0 / W2, 0.0)
    qw_ref[...] = _dot(pool, q)
    kw_ref[...] = _dot(pool, kv[:, :QK])


def _route_kernel(qw_ref, kw_ref, idx_ref, wgt_ref):
    qw = qw_ref[...]
    kw = kw_ref[...]
    logit = _dot_t(qw * SCALE, kw)
    col = jax.lax.broadcasted_iota(jnp.int32, (P2, P2), 1)
    lane = jax.lax.broadcasted_iota(jnp.int32, (P2, 128), 1)
    idx_out = jnp.zeros((P2, 128), jnp.int32)
    val_out = jnp.zeros((P2, 128), jnp.float32)
    cur = logit
    for t in range(TOPK):
        m = jnp.max(cur, axis=-1, keepdims=True)
        a = jnp.min(jnp.where(cur == m, col, P2), axis=-1, keepdims=True)
        idx_out = jnp.where(lane == t, a, idx_out)
        val_out = jnp.where(lane == t, m, val_out)
        cur = jnp.where(col == a, -jnp.inf, cur)
    e = jnp.where(lane < TOPK, jnp.exp(val_out - val_out[:, :1]), 0.0)
    s = jnp.sum(e, axis=-1, keepdims=True)
    idx_ref[...] = idx_out
    wgt_ref[...] = e / s


def _attn_kernel(ridx_ref, q_ref, kv0_ref, kv1_ref, kv2_ref, kv3_ref,
                 rw_ref, o_ref):
    kv_refs = (kv0_ref, kv1_ref, kv2_ref, kv3_ref)
    q = q_ref[...] * SCALE
    lane = jax.lax.broadcasted_iota(jnp.int32, (1, 128), 1)
    rw = rw_ref[0]
    wts = [jnp.sum(jnp.where(lane == t, rw, 0.0)) for t in range(TOPK)]
    ls = [_dot_t(q, kv_refs[t][:, :QK]) * wts[t] for t in range(TOPK)]
    l = jnp.concatenate(ls, axis=1)
    m = jnp.max(l, axis=-1, keepdims=True)
    p = jnp.exp(l - m)
    s = jnp.sum(p, axis=-1, keepdims=True)
    acc = _dot(p[:, :W2] * wts[0], kv0_ref[:, QK:])
    for t in range(1, TOPK):
        acc += _dot(p[:, t * W2:(t + 1) * W2] * wts[t], kv_refs[t][:, QK:])
    o_ref[...] = acc / s


def _proj_kernel(o_ref, w_ref, b_ref, y_ref):
    y_ref[...] = _dot(o_ref[...], w_ref[...]) + b_ref[...]


def kernel(x, W_qkv, b_qkv, W_o, b_o):
    n, p2, w2, dim = x.shape
    rows = n * p2 * w2
    x2 = x.reshape(rows, dim)
    b2 = b_qkv.reshape(1, 2 * QK + DIM)

    q2, kv2, qw, kw = pl.pallas_call(
        _qkv_kernel,
        grid=(rows // (PB * W2),),
        in_specs=[
            pl.BlockSpec((PB * W2, DIM), lambda g: (g, 0)),
            pl.BlockSpec((DIM, 2 * QK + DIM), lambda g: (0, 0)),
            pl.BlockSpec((1, 2 * QK + DIM), lambda g: (0, 0)),
        ],
        out_specs=[
            pl.BlockSpec((PB * W2, QK), lambda g: (g, 0)),
            pl.BlockSpec((PB * W2, KV), lambda g: (g, 0)),
            pl.BlockSpec((PB, QK), lambda g: (g, 0)),
            pl.BlockSpec((PB, QK), lambda g: (g, 0)),
        ],
        out_shape=[
            jax.ShapeDtypeStruct((rows, QK), jnp.float32),
            jax.ShapeDtypeStruct((rows, KV), jnp.float32),
            jax.ShapeDtypeStruct((n * p2, QK), jnp.float32),
            jax.ShapeDtypeStruct((n * p2, QK), jnp.float32),
        ],
    )(x2, W_qkv, b2)

    r_idx, r_wgt = pl.pallas_call(
        _route_kernel,
        grid=(n,),
        in_specs=[
            pl.BlockSpec((P2, QK), lambda b: (b, 0)),
            pl.BlockSpec((P2, QK), lambda b: (b, 0)),
        ],
        out_specs=[
            pl.BlockSpec((P2, 128), lambda b: (b, 0)),
            pl.BlockSpec((P2, 128), lambda b: (b, 0)),
        ],
        out_shape=[
            jax.ShapeDtypeStruct((n * p2, 128), jnp.int32),
            jax.ShapeDtypeStruct((n * p2, 128), jnp.float32),
        ],
    )(qw, kw)

    rw3 = r_wgt.reshape(n * p2, 1, 128)
    bo2 = b_o.reshape(1, DIM)

    def _kv_spec(t):
        return pl.BlockSpec(
            (W2, KV), lambda i, ridx: ((i // P2) * P2 + ridx[i, t], 0))

    attn = pl.pallas_call(
        _attn_kernel,
        grid_spec=pltpu.PrefetchScalarGridSpec(
            num_scalar_prefetch=1,
            grid=(n * p2,),
            in_specs=[
                pl.BlockSpec((W2, QK), lambda i, ridx: (i, 0)),
                _kv_spec(0), _kv_spec(1), _kv_spec(2), _kv_spec(3),
                pl.BlockSpec((1, 1, 128), lambda i, ridx: (i, 0, 0)),
            ],
            out_specs=pl.BlockSpec((W2, DIM), lambda i, ridx: (i, 0)),
        ),
        out_shape=jax.ShapeDtypeStruct((rows, DIM), jnp.float32),
        compiler_params=pltpu.CompilerParams(
            dimension_semantics=("arbitrary",)),
    )(r_idx, q2, kv2, kv2, kv2, kv2, rw3)

    out = pl.pallas_call(
        _proj_kernel,
        grid=(rows // (PB * W2),),
        in_specs=[
            pl.BlockSpec((PB * W2, DIM), lambda g: (g, 0)),
            pl.BlockSpec((DIM, DIM), lambda g: (0, 0)),
            pl.BlockSpec((1, DIM), lambda g: (0, 0)),
        ],
        out_specs=pl.BlockSpec((PB * W2, DIM), lambda g: (g, 0)),
        out_shape=jax.ShapeDtypeStruct((rows, DIM), jnp.float32),
    )(attn, W_o, bo2)

    return out.reshape(n, p2, w2, dim)


# speedup vs baseline: 2.2535x; 1.0571x over previous
import jax
import jax.numpy as jnp
from jax.experimental import pallas as pl
from jax.experimental.pallas import tpu as pltpu

N = 2
P2 = 64
W2 = 64
DIM = 1024
QK = 1024
KV = 2048
TOPK = 4
SCALE = QK ** -0.5
PB = 8


def _dot(a, b):
    return jax.lax.dot_general(a, b, (((1,), (0,)), ((), ())),
                               preferred_element_type=jnp.float32)


def _dot_t(a, b):
    return jax.lax.dot_general(a, b, (((1,), (1,)), ((), ())),
                               preferred_element_type=jnp.float32)


def _qkv_kernel(x_ref, w_ref, b_ref, q_ref, kv_ref, xm_ref):
    x = x_ref[...]
    r = jax.lax.broadcasted_iota(jnp.int32, (PB, PB * W2), 0)
    c = jax.lax.broadcasted_iota(jnp.int32, (PB, PB * W2), 1) // W2
    pool = jnp.where(r == c, 1.0 / W2, 0.0)
    xm_ref[...] = _dot(pool, x)
    qkv = _dot(x.astype(jnp.bfloat16), w_ref[...]) + b_ref[...]
    q_ref[...] = qkv[:, :QK].astype(jnp.bfloat16)
    kv_ref[...] = qkv[:, QK:].astype(jnp.bfloat16)


def _route_kernel(xm_ref, wq_ref, wk_ref, bq_ref, bk_ref, idx_ref, wgt_ref):
    xm = xm_ref[...]
    qw = _dot(xm, wq_ref[...]) + bq_ref[...]
    kw = _dot(xm, wk_ref[...]) + bk_ref[...]
    logit = _dot_t(qw * SCALE, kw)
    col = jax.lax.broadcasted_iota(jnp.int32, (P2, P2), 1)
    lane = jax.lax.broadcasted_iota(jnp.int32, (P2, 128), 1)
    idx_out = jnp.zeros((P2, 128), jnp.int32)
    val_out = jnp.zeros((P2, 128), jnp.float32)
    cur = logit
    for t in range(TOPK):
        m = jnp.max(cur, axis=-1, keepdims=True)
        a = jnp.min(jnp.where(cur == m, col, P2), axis=-1, keepdims=True)
        idx_out = jnp.where(lane == t, a, idx_out)
        val_out = jnp.where(lane == t, m, val_out)
        cur = jnp.where(col == a, -jnp.inf, cur)
    e = jnp.where(lane < TOPK, jnp.exp(val_out - val_out[:, :1]), 0.0)
    s = jnp.sum(e, axis=-1, keepdims=True)
    idx_ref[...] = idx_out
    wgt_ref[...] = e / s


def _attn_kernel(ridx_ref, q_ref, kv0_ref, kv1_ref, kv2_ref, kv3_ref,
                 rw_ref, o_ref):
    kv_refs = (kv0_ref, kv1_ref, kv2_ref, kv3_ref)
    q = q_ref[...]
    lane = jax.lax.broadcasted_iota(jnp.int32, (1, 128), 1)
    rw = rw_ref[0]
    wts = [jnp.sum(jnp.where(lane == t, rw, 0.0)) for t in range(TOPK)]
    ls = [_dot_t(q, kv_refs[t][:, :QK]) * (wts[t] * SCALE)
          for t in range(TOPK)]
    l = jnp.concatenate(ls, axis=1)
    m = jnp.max(l, axis=-1, keepdims=True)
    p = jnp.exp(l - m)
    s = jnp.sum(p, axis=-1, keepdims=True)
    acc = _dot((p[:, :W2] * wts[0]).astype(jnp.bfloat16), kv0_ref[:, QK:])
    for t in range(1, TOPK):
        pt = (p[:, t * W2:(t + 1) * W2] * wts[t]).astype(jnp.bfloat16)
        acc += _dot(pt, kv_refs[t][:, QK:])
    o_ref[...] = (acc / s).astype(jnp.bfloat16)


def _proj_kernel(o_ref, w_ref, b_ref, y_ref):
    y_ref[...] = _dot(o_ref[...], w_ref[...]) + b_ref[...]


def kernel(x, W_qkv, b_qkv, W_o, b_o):
    n, p2, w2, dim = x.shape
    rows = n * p2 * w2
    x2 = x.reshape(rows, dim)
    b2 = b_qkv.reshape(1, 2 * QK + DIM)

    q2, kv2, xm = pl.pallas_call(
        _qkv_kernel,
        grid=(rows // (PB * W2),),
        in_specs=[
            pl.BlockSpec((PB * W2, DIM), lambda g: (g, 0)),
            pl.BlockSpec((DIM, 2 * QK + DIM), lambda g: (0, 0)),
            pl.BlockSpec((1, 2 * QK + DIM), lambda g: (0, 0)),
        ],
        out_specs=[
            pl.BlockSpec((PB * W2, QK), lambda g: (g, 0)),
            pl.BlockSpec((PB * W2, KV), lambda g: (g, 0)),
            pl.BlockSpec((PB, DIM), lambda g: (g, 0)),
        ],
        out_shape=[
            jax.ShapeDtypeStruct((rows, QK), jnp.bfloat16),
            jax.ShapeDtypeStruct((rows, KV), jnp.bfloat16),
            jax.ShapeDtypeStruct((n * p2, DIM), jnp.float32),
        ],
    )(x2, W_qkv.astype(jnp.bfloat16), b2)

    r_idx, r_wgt = pl.pallas_call(
        _route_kernel,
        grid=(n,),
        in_specs=[
            pl.BlockSpec((P2, DIM), lambda b: (b, 0)),
            pl.BlockSpec((DIM, QK), lambda b: (0, 0)),
            pl.BlockSpec((DIM, QK), lambda b: (0, 0)),
            pl.BlockSpec((1, QK), lambda b: (0, 0)),
            pl.BlockSpec((1, QK), lambda b: (0, 0)),
        ],
        out_specs=[
            pl.BlockSpec((P2, 128), lambda b: (b, 0)),
            pl.BlockSpec((P2, 128), lambda b: (b, 0)),
        ],
        out_shape=[
            jax.ShapeDtypeStruct((n * p2, 128), jnp.int32),
            jax.ShapeDtypeStruct((n * p2, 128), jnp.float32),
        ],
    )(xm, W_qkv[:, :QK], W_qkv[:, QK:2 * QK],
      b_qkv[:QK].reshape(1, QK), b_qkv[QK:2 * QK].reshape(1, QK))

    rw3 = r_wgt.reshape(n * p2, 1, 128)
    bo2 = b_o.reshape(1, DIM)

    def _kv_spec(t):
        return pl.BlockSpec(
            (W2, KV), lambda i, ridx: ((i // P2) * P2 + ridx[i, t], 0))

    attn = pl.pallas_call(
        _attn_kernel,
        grid_spec=pltpu.PrefetchScalarGridSpec(
            num_scalar_prefetch=1,
            grid=(n * p2,),
            in_specs=[
                pl.BlockSpec((W2, QK), lambda i, ridx: (i, 0)),
                _kv_spec(0), _kv_spec(1), _kv_spec(2), _kv_spec(3),
                pl.BlockSpec((1, 1, 128), lambda i, ridx: (i, 0, 0)),
            ],
            out_specs=pl.BlockSpec((W2, DIM), lambda i, ridx: (i, 0)),
        ),
        out_shape=jax.ShapeDtypeStruct((rows, DIM), jnp.bfloat16),
        compiler_params=pltpu.CompilerParams(
            dimension_semantics=("arbitrary",)),
    )(r_idx, q2, kv2, kv2, kv2, kv2, rw3)

    out = pl.pallas_call(
        _proj_kernel,
        grid=(rows // (PB * W2),),
        in_specs=[
            pl.BlockSpec((PB * W2, DIM), lambda g: (g, 0)),
            pl.BlockSpec((DIM, DIM), lambda g: (0, 0)),
            pl.BlockSpec((1, DIM), lambda g: (0, 0)),
        ],
        out_specs=pl.BlockSpec((PB * W2, DIM), lambda g: (g, 0)),
        out_shape=jax.ShapeDtypeStruct((rows, DIM), jnp.float32),
    )(attn, W_o.astype(jnp.bfloat16), bo2)

    return out.reshape(n, p2, w2, dim)


# speedup vs baseline: 2.4435x; 1.0843x over previous
import jax
import jax.numpy as jnp
from jax.experimental import pallas as pl
from jax.experimental.pallas import tpu as pltpu

N = 2
P2 = 64
W2 = 64
DIM = 1024
QK = 1024
KV = 2048
TOPK = 4
SCALE = QK ** -0.5
PB = 8


def _dot(a, b, precision=None):
    return jax.lax.dot_general(a, b, (((1,), (0,)), ((), ())),
                               preferred_element_type=jnp.float32,
                               precision=precision)


def _dot_t(a, b):
    return jax.lax.dot_general(a, b, (((1,), (1,)), ((), ())),
                               preferred_element_type=jnp.float32)


def _qkv_kernel(x_ref, w_ref, b_ref, q_ref, kv_ref, qw_ref, kw_ref):
    x = x_ref[...]
    qkv = _dot(x, w_ref[...]) + b_ref[...]
    q = qkv[:, :QK]
    kv = qkv[:, QK:]
    q_ref[...] = q.astype(jnp.bfloat16)
    kv_ref[...] = kv.astype(jnp.bfloat16)
    qw_ref[...] = jnp.mean(q.reshape(PB, W2, QK), axis=1)
    kw_ref[...] = jnp.mean(kv[:, :QK].reshape(PB, W2, QK), axis=1)


def _route_kernel(qw_ref, kw_ref, idx_ref, wgt_ref):
    qw = qw_ref[...] * SCALE
    kw = kw_ref[...]
    logit = _dot_t(qw, kw)
    col = jax.lax.broadcasted_iota(jnp.int32, (P2, P2), 1)
    lane = jax.lax.broadcasted_iota(jnp.int32, (P2, 128), 1)
    idx_out = jnp.zeros((P2, 128), jnp.int32)
    val_out = jnp.zeros((P2, 128), jnp.float32)
    cur = logit
    for t in range(TOPK):
        m = jnp.max(cur, axis=-1, keepdims=True)
        a = jnp.min(jnp.where(cur == m, col, P2), axis=-1, keepdims=True)
        idx_out = jnp.where(lane == t, a, idx_out)
        val_out = jnp.where(lane == t, m, val_out)
        cur = jnp.where(col == a, -jnp.inf, cur)
    e = jnp.where(lane < TOPK, jnp.exp(val_out - val_out[:, :1]), 0.0)
    s = jnp.sum(e, axis=-1, keepdims=True)
    idx_ref[...] = idx_out
    wgt_ref[...] = e / s


def _attn_kernel(ridx_ref, q_ref, kv0_ref, kv1_ref, kv2_ref, kv3_ref,
                 rw_ref, o_ref):
    kv_refs = (kv0_ref, kv1_ref, kv2_ref, kv3_ref)
    q = q_ref[...]
    lane = jax.lax.broadcasted_iota(jnp.int32, (1, 128), 1)
    rw = rw_ref[0]
    wts = [jnp.sum(jnp.where(lane == t, rw, 0.0)) for t in range(TOPK)]
    ls = [_dot_t(q, kv_refs[t][:, :QK]) * (wts[t] * SCALE)
          for t in range(TOPK)]
    l = jnp.concatenate(ls, axis=1)
    m = jnp.max(l, axis=-1, keepdims=True)
    p = jnp.exp(l - m)
    s = jnp.sum(p, axis=-1, keepdims=True)
    acc = _dot((p[:, :W2] * wts[0]).astype(jnp.bfloat16), kv0_ref[:, QK:])
    for t in range(1, TOPK):
        pt = (p[:, t * W2:(t + 1) * W2] * wts[t]).astype(jnp.bfloat16)
        acc += _dot(pt, kv_refs[t][:, QK:])
    o_ref[...] = (acc / s).astype(jnp.bfloat16)


def _proj_kernel(o_ref, w_ref, b_ref, y_ref):
    y_ref[...] = _dot(o_ref[...], w_ref[...]) + b_ref[...]


def kernel(x, W_qkv, b_qkv, W_o, b_o):
    n, p2, w2, dim = x.shape
    rows = n * p2 * w2
    x2 = x.reshape(rows, dim)
    b2 = b_qkv.reshape(1, 2 * QK + DIM)

    q2, kv2, qw, kw = pl.pallas_call(
        _qkv_kernel,
        grid=(rows // (PB * W2),),
        in_specs=[
            pl.BlockSpec((PB * W2, DIM), lambda g: (g, 0)),
            pl.BlockSpec((DIM, 2 * QK + DIM), lambda g: (0, 0)),
            pl.BlockSpec((1, 2 * QK + DIM), lambda g: (0, 0)),
        ],
        out_specs=[
            pl.BlockSpec((PB * W2, QK), lambda g: (g, 0)),
            pl.BlockSpec((PB * W2, KV), lambda g: (g, 0)),
            pl.BlockSpec((PB, QK), lambda g: (g, 0)),
            pl.BlockSpec((PB, QK), lambda g: (g, 0)),
        ],
        out_shape=[
            jax.ShapeDtypeStruct((rows, QK), jnp.bfloat16),
            jax.ShapeDtypeStruct((rows, KV), jnp.bfloat16),
            jax.ShapeDtypeStruct((n * p2, QK), jnp.float32),
            jax.ShapeDtypeStruct((n * p2, QK), jnp.float32),
        ],
    )(x2, W_qkv, b2)

    r_idx, r_wgt = pl.pallas_call(
        _route_kernel,
        grid=(n,),
        in_specs=[
            pl.BlockSpec((P2, QK), lambda b: (b, 0)),
            pl.BlockSpec((P2, QK), lambda b: (b, 0)),
        ],
        out_specs=[
            pl.BlockSpec((P2, 128), lambda b: (b, 0)),
            pl.BlockSpec((P2, 128), lambda b: (b, 0)),
        ],
        out_shape=[
            jax.ShapeDtypeStruct((n * p2, 128), jnp.int32),
            jax.ShapeDtypeStruct((n * p2, 128), jnp.float32),
        ],
    )(qw, kw)

    rw3 = r_wgt.reshape(n * p2, 1, 128)
    bo2 = b_o.reshape(1, DIM)

    def _kv_spec(t):
        return pl.BlockSpec(
            (W2, KV), lambda i, ridx: ((i // P2) * P2 + ridx[i, t], 0))

    attn = pl.pallas_call(
        _attn_kernel,
        grid_spec=pltpu.PrefetchScalarGridSpec(
            num_scalar_prefetch=1,
            grid=(n * p2,),
            in_specs=[
                pl.BlockSpec((W2, QK), lambda i, ridx: (i, 0)),
                _kv_spec(0), _kv_spec(1), _kv_spec(2), _kv_spec(3),
                pl.BlockSpec((1, 1, 128), lambda i, ridx: (i, 0, 0)),
            ],
            out_specs=pl.BlockSpec((W2, DIM), lambda i, ridx: (i, 0)),
        ),
        out_shape=jax.ShapeDtypeStruct((rows, DIM), jnp.bfloat16),
        compiler_params=pltpu.CompilerParams(
            dimension_semantics=("arbitrary",)),
    )(r_idx, q2, kv2, kv2, kv2, kv2, rw3)

    out = pl.pallas_call(
        _proj_kernel,
        grid=(rows // (PB * W2),),
        in_specs=[
            pl.BlockSpec((PB * W2, DIM), lambda g: (g, 0)),
            pl.BlockSpec((DIM, DIM), lambda g: (0, 0)),
            pl.BlockSpec((1, DIM), lambda g: (0, 0)),
        ],
        out_specs=pl.BlockSpec((PB * W2, DIM), lambda g: (g, 0)),
        out_shape=jax.ShapeDtypeStruct((rows, DIM), jnp.float32),
    )(attn, W_o.astype(jnp.bfloat16), bo2)

    return out.reshape(n, p2, w2, dim)


# speedup vs baseline: 2.5480x; 1.0427x over previous
import jax
import jax.numpy as jnp
from jax.experimental import pallas as pl
from jax.experimental.pallas import tpu as pltpu

N = 2
P2 = 64
W2 = 64
DIM = 1024
QK = 1024
KV = 2048
TOPK = 4
SCALE = QK ** -0.5
PB = 8


def _dot(a, b, precision=None):
    return jax.lax.dot_general(a, b, (((1,), (0,)), ((), ())),
                               preferred_element_type=jnp.float32,
                               precision=precision)


def _dot_t(a, b):
    return jax.lax.dot_general(a, b, (((1,), (1,)), ((), ())),
                               preferred_element_type=jnp.float32)


def _qkv_kernel(x_ref, w_ref, b_ref, q_ref, kv_ref, qw_ref, kw_ref):
    x = x_ref[...]
    qkv = _dot(x, w_ref[...]) + b_ref[...]
    q = qkv[:, :QK]
    kv = qkv[:, QK:]
    q_ref[...] = q.astype(jnp.bfloat16)
    kv_ref[...] = kv.astype(jnp.bfloat16)
    qw_ref[...] = jnp.mean(q.reshape(PB, W2, QK), axis=1)
    kw_ref[...] = jnp.mean(kv[:, :QK].reshape(PB, W2, QK), axis=1)


def _route_kernel(qw_ref, kw_ref, idx_ref, wgt_ref):
    qw = qw_ref[...] * SCALE
    kw = kw_ref[...]
    logit = _dot_t(qw, kw)
    col = jax.lax.broadcasted_iota(jnp.int32, (P2, P2), 1)
    lane = jax.lax.broadcasted_iota(jnp.int32, (P2, 128), 1)
    idx_out = jnp.zeros((P2, 128), jnp.int32)
    val_out = jnp.zeros((P2, 128), jnp.float32)
    cur = logit
    for t in range(TOPK):
        m = jnp.max(cur, axis=-1, keepdims=True)
        a = jnp.min(jnp.where(cur == m, col, P2), axis=-1, keepdims=True)
        idx_out = jnp.where(lane == t, a, idx_out)
        val_out = jnp.where(lane == t, m, val_out)
        cur = jnp.where(col == a, -jnp.inf, cur)
    e = jnp.where(lane < TOPK, jnp.exp(val_out - val_out[:, :1]), 0.0)
    s = jnp.sum(e, axis=-1, keepdims=True)
    idx_ref[...] = idx_out
    wgt_ref[...] = e / s


def _attn_kernel(ridx_ref, q_ref, kv_ref, rw_ref, o_ref):
    b = pl.program_id(0)
    j = pl.program_id(1)
    row = b * P2 + j
    q = q_ref[...]
    lane = jax.lax.broadcasted_iota(jnp.int32, (1, 128), 1)
    rw = rw_ref[0]
    wts = [jnp.sum(jnp.where(lane == t, rw, 0.0)) for t in range(TOPK)]
    kvt = [kv_ref[0, pl.ds(ridx_ref[row, t] * W2, W2), :] for t in range(TOPK)]
    ls = [_dot_t(q, kvt[t][:, :QK]) * (wts[t] * SCALE) for t in range(TOPK)]
    l = jnp.concatenate(ls, axis=1)
    m = jnp.max(l, axis=-1, keepdims=True)
    p = jnp.exp(l - m)
    s = jnp.sum(p, axis=-1, keepdims=True)
    acc = _dot((p[:, :W2] * wts[0]).astype(jnp.bfloat16), kvt[0][:, QK:])
    for t in range(1, TOPK):
        pt = (p[:, t * W2:(t + 1) * W2] * wts[t]).astype(jnp.bfloat16)
        acc += _dot(pt, kvt[t][:, QK:])
    o_ref[...] = (acc / s).astype(jnp.bfloat16)


def _proj_kernel(o_ref, w_ref, b_ref, y_ref):
    y_ref[...] = _dot(o_ref[...], w_ref[...]) + b_ref[...]


def kernel(x, W_qkv, b_qkv, W_o, b_o):
    n, p2, w2, dim = x.shape
    rows = n * p2 * w2
    x2 = x.reshape(rows, dim)
    b2 = b_qkv.reshape(1, 2 * QK + DIM)

    q2, kv2, qw, kw = pl.pallas_call(
        _qkv_kernel,
        grid=(rows // (PB * W2),),
        in_specs=[
            pl.BlockSpec((PB * W2, DIM), lambda g: (g, 0)),
            pl.BlockSpec((DIM, 2 * QK + DIM), lambda g: (0, 0)),
            pl.BlockSpec((1, 2 * QK + DIM), lambda g: (0, 0)),
        ],
        out_specs=[
            pl.BlockSpec((PB * W2, QK), lambda g: (g, 0)),
            pl.BlockSpec((PB * W2, KV), lambda g: (g, 0)),
            pl.BlockSpec((PB, QK), lambda g: (g, 0)),
            pl.BlockSpec((PB, QK), lambda g: (g, 0)),
        ],
        out_shape=[
            jax.ShapeDtypeStruct((rows, QK), jnp.bfloat16),
            jax.ShapeDtypeStruct((rows, KV), jnp.bfloat16),
            jax.ShapeDtypeStruct((n * p2, QK), jnp.float32),
            jax.ShapeDtypeStruct((n * p2, QK), jnp.float32),
        ],
    )(x2, W_qkv, b2)

    r_idx, r_wgt = pl.pallas_call(
        _route_kernel,
        grid=(n,),
        in_specs=[
            pl.BlockSpec((P2, QK), lambda b: (b, 0)),
            pl.BlockSpec((P2, QK), lambda b: (b, 0)),
        ],
        out_specs=[
            pl.BlockSpec((P2, 128), lambda b: (b, 0)),
            pl.BlockSpec((P2, 128), lambda b: (b, 0)),
        ],
        out_shape=[
            jax.ShapeDtypeStruct((n * p2, 128), jnp.int32),
            jax.ShapeDtypeStruct((n * p2, 128), jnp.float32),
        ],
    )(qw, kw)

    rw3 = r_wgt.reshape(n * p2, 1, 128)
    bo2 = b_o.reshape(1, DIM)

    attn = pl.pallas_call(
        _attn_kernel,
        grid_spec=pltpu.PrefetchScalarGridSpec(
            num_scalar_prefetch=1,
            grid=(n, p2),
            in_specs=[
                pl.BlockSpec((W2, QK), lambda b, j, ridx: (b * P2 + j, 0)),
                pl.BlockSpec((1, P2 * W2, KV), lambda b, j, ridx: (b, 0, 0)),
                pl.BlockSpec((1, 1, 128), lambda b, j, ridx: (b * P2 + j, 0, 0)),
            ],
            out_specs=pl.BlockSpec((W2, DIM), lambda b, j, ridx: (b * P2 + j, 0)),
        ),
        out_shape=jax.ShapeDtypeStruct((rows, DIM), jnp.bfloat16),
        compiler_params=pltpu.CompilerParams(
            dimension_semantics=("arbitrary", "arbitrary"),
            vmem_limit_bytes=100 * 1024 * 1024),
    )(r_idx, q2, kv2.reshape(n, p2 * w2, KV), rw3)

    out = pl.pallas_call(
        _proj_kernel,
        grid=(rows // (PB * W2),),
        in_specs=[
            pl.BlockSpec((PB * W2, DIM), lambda g: (g, 0)),
            pl.BlockSpec((DIM, DIM), lambda g: (0, 0)),
            pl.BlockSpec((1, DIM), lambda g: (0, 0)),
        ],
        out_specs=pl.BlockSpec((PB * W2, DIM), lambda g: (g, 0)),
        out_shape=jax.ShapeDtypeStruct((rows, DIM), jnp.float32),
    )(attn, W_o.astype(jnp.bfloat16), bo2)

    return out.reshape(n, p2, w2, dim)


# speedup vs baseline: 3.1761x; 1.2465x over previous
import jax
import jax.numpy as jnp
from jax.experimental import pallas as pl
from jax.experimental.pallas import tpu as pltpu

N = 2
P2 = 64
W2 = 64
DIM = 1024
QK = 1024
KV = 2048
TOPK = 4
SCALE = QK ** -0.5
PB = 8


def _dot(a, b, precision=None):
    return jax.lax.dot_general(a, b, (((1,), (0,)), ((), ())),
                               preferred_element_type=jnp.float32,
                               precision=precision)


def _dot_t(a, b):
    return jax.lax.dot_general(a, b, (((1,), (1,)), ((), ())),
                               preferred_element_type=jnp.float32)


def _qkv_kernel(x_ref, w_ref, b_ref, q_ref, kv_ref, qw_ref, kw_ref):
    x = x_ref[...]
    qkv = _dot(x, w_ref[...]) + b_ref[...]
    q = qkv[:, :QK]
    kv = qkv[:, QK:]
    q_ref[...] = q.astype(jnp.bfloat16)
    kv_ref[...] = kv.astype(jnp.bfloat16)
    qw_ref[...] = jnp.mean(q.reshape(PB, W2, QK), axis=1)
    kw_ref[...] = jnp.mean(kv[:, :QK].reshape(PB, W2, QK), axis=1)


def _route_kernel(qw_ref, kw_ref, idx_ref, wgt_ref):
    qw = qw_ref[...] * SCALE
    kw = kw_ref[...]
    logit = _dot_t(qw, kw)
    col = jax.lax.broadcasted_iota(jnp.int32, (P2, P2), 1)
    lane = jax.lax.broadcasted_iota(jnp.int32, (P2, 128), 1)
    idx_out = jnp.zeros((P2, 128), jnp.int32)
    val_out = jnp.zeros((P2, 128), jnp.float32)
    cur = logit
    for t in range(TOPK):
        m = jnp.max(cur, axis=-1, keepdims=True)
        a = jnp.min(jnp.where(cur == m, col, P2), axis=-1, keepdims=True)
        idx_out = jnp.where(lane == t, a, idx_out)
        val_out = jnp.where(lane == t, m, val_out)
        cur = jnp.where(col == a, -jnp.inf, cur)
    e = jnp.where(lane < TOPK, jnp.exp(val_out - val_out[:, :1]), 0.0)
    s = jnp.sum(e, axis=-1, keepdims=True)
    idx_ref[...] = idx_out
    wgt_ref[...] = e / s


G = 8


def _attn_kernel(ridx_ref, q_ref, kv_ref, rw_ref, wo_ref, bo_ref, o_ref,
                 acc_ref):
    b = pl.program_id(0)
    jj = pl.program_id(1)
    lane = jax.lax.broadcasted_iota(jnp.int32, (1, 128), 1)
    for w in range(G):
        row = b * P2 + jj * G + w
        q = q_ref[w * W2:(w + 1) * W2, :]
        rww = rw_ref[w:w + 1, :]
        wts = [jnp.sum(jnp.where(lane == t, rww, 0.0)) for t in range(TOPK)]
        kvt = [kv_ref[0, pl.ds(ridx_ref[row, t] * W2, W2), :]
               for t in range(TOPK)]
        ls = [_dot_t(q, kvt[t][:, :QK]) * (wts[t] * SCALE)
              for t in range(TOPK)]
        l = jnp.concatenate(ls, axis=1)
        m = jnp.max(l, axis=-1, keepdims=True)
        p = jnp.exp(l - m)
        s = jnp.sum(p, axis=-1, keepdims=True)
        acc = _dot((p[:, :W2] * wts[0]).astype(jnp.bfloat16), kvt[0][:, QK:])
        for t in range(1, TOPK):
            pt = (p[:, t * W2:(t + 1) * W2] * wts[t]).astype(jnp.bfloat16)
            acc += _dot(pt, kvt[t][:, QK:])
        acc_ref[w * W2:(w + 1) * W2, :] = (acc / s).astype(jnp.bfloat16)
    o_ref[...] = _dot(acc_ref[...], wo_ref[...]) + bo_ref[...]


def kernel(x, W_qkv, b_qkv, W_o, b_o):
    n, p2, w2, dim = x.shape
    rows = n * p2 * w2
    x2 = x.reshape(rows, dim)
    b2 = b_qkv.reshape(1, 2 * QK + DIM)

    q2, kv2, qw, kw = pl.pallas_call(
        _qkv_kernel,
        grid=(rows // (PB * W2),),
        in_specs=[
            pl.BlockSpec((PB * W2, DIM), lambda g: (g, 0)),
            pl.BlockSpec((DIM, 2 * QK + DIM), lambda g: (0, 0)),
            pl.BlockSpec((1, 2 * QK + DIM), lambda g: (0, 0)),
        ],
        out_specs=[
            pl.BlockSpec((PB * W2, QK), lambda g: (g, 0)),
            pl.BlockSpec((PB * W2, KV), lambda g: (g, 0)),
            pl.BlockSpec((PB, QK), lambda g: (g, 0)),
            pl.BlockSpec((PB, QK), lambda g: (g, 0)),
        ],
        out_shape=[
            jax.ShapeDtypeStruct((rows, QK), jnp.bfloat16),
            jax.ShapeDtypeStruct((rows, KV), jnp.bfloat16),
            jax.ShapeDtypeStruct((n * p2, QK), jnp.float32),
            jax.ShapeDtypeStruct((n * p2, QK), jnp.float32),
        ],
    )(x2, W_qkv, b2)

    r_idx, r_wgt = pl.pallas_call(
        _route_kernel,
        grid=(n,),
        in_specs=[
            pl.BlockSpec((P2, QK), lambda b: (b, 0)),
            pl.BlockSpec((P2, QK), lambda b: (b, 0)),
        ],
        out_specs=[
            pl.BlockSpec((P2, 128), lambda b: (b, 0)),
            pl.BlockSpec((P2, 128), lambda b: (b, 0)),
        ],
        out_shape=[
            jax.ShapeDtypeStruct((n * p2, 128), jnp.int32),
            jax.ShapeDtypeStruct((n * p2, 128), jnp.float32),
        ],
    )(qw, kw)

    bo2 = b_o.reshape(1, DIM)

    out = pl.pallas_call(
        _attn_kernel,
        grid_spec=pltpu.PrefetchScalarGridSpec(
            num_scalar_prefetch=1,
            grid=(n, p2 // G),
            in_specs=[
                pl.BlockSpec((G * W2, QK),
                             lambda b, jj, ridx: (b * (P2 // G) + jj, 0)),
                pl.BlockSpec((1, P2 * W2, KV), lambda b, jj, ridx: (b, 0, 0)),
                pl.BlockSpec((G, 128),
                             lambda b, jj, ridx: (b * (P2 // G) + jj, 0)),
                pl.BlockSpec((DIM, DIM), lambda b, jj, ridx: (0, 0)),
                pl.BlockSpec((1, DIM), lambda b, jj, ridx: (0, 0)),
            ],
            out_specs=pl.BlockSpec(
                (G * W2, DIM), lambda b, jj, ridx: (b * (P2 // G) + jj, 0)),
            scratch_shapes=[pltpu.VMEM((G * W2, DIM), jnp.bfloat16)],
        ),
        out_shape=jax.ShapeDtypeStruct((rows, DIM), jnp.float32),
        compiler_params=pltpu.CompilerParams(
            dimension_semantics=("arbitrary", "arbitrary"),
            vmem_limit_bytes=100 * 1024 * 1024),
    )(r_idx, q2, kv2.reshape(n, p2 * w2, KV), r_wgt,
      W_o.astype(jnp.bfloat16), bo2)

    return out.reshape(n, p2, w2, dim)


# speedup vs baseline: 3.1889x; 1.0040x over previous
import jax
import jax.numpy as jnp
from jax.experimental import pallas as pl
from jax.experimental.pallas import tpu as pltpu

N = 2
P2 = 64
W2 = 64
DIM = 1024
QK = 1024
KV = 2048
TOPK = 4
SCALE = QK ** -0.5
PB = 8


def _dot(a, b, precision=None):
    return jax.lax.dot_general(a, b, (((1,), (0,)), ((), ())),
                               preferred_element_type=jnp.float32,
                               precision=precision)


def _dot_t(a, b):
    return jax.lax.dot_general(a, b, (((1,), (1,)), ((), ())),
                               preferred_element_type=jnp.float32)


def _qkv_kernel(x_ref, w_ref, b_ref, q_ref, kv_ref, idx_ref, wgt_ref,
                qw_s, kw_s):
    g = pl.program_id(0)
    gb = P2 // PB
    x = x_ref[...]
    qkv = _dot(x, w_ref[...]) + b_ref[...]
    q = qkv[:, :QK]
    kv = qkv[:, QK:]
    q_ref[...] = q.astype(jnp.bfloat16)
    kv_ref[...] = kv.astype(jnp.bfloat16)
    r = (g % gb) * PB
    qw_s[pl.ds(r, PB), :] = jnp.mean(q.reshape(PB, W2, QK), axis=1)
    kw_s[pl.ds(r, PB), :] = jnp.mean(kv[:, :QK].reshape(PB, W2, QK), axis=1)

    @pl.when(g % gb == gb - 1)
    def _():
        logit = _dot_t(qw_s[...] * SCALE, kw_s[...])
        col = jax.lax.broadcasted_iota(jnp.int32, (P2, P2), 1)
        lane = jax.lax.broadcasted_iota(jnp.int32, (P2, 128), 1)
        idx_out = jnp.zeros((P2, 128), jnp.int32)
        val_out = jnp.zeros((P2, 128), jnp.float32)
        cur = logit
        for t in range(TOPK):
            m = jnp.max(cur, axis=-1, keepdims=True)
            a = jnp.min(jnp.where(cur == m, col, P2), axis=-1, keepdims=True)
            idx_out = jnp.where(lane == t, a, idx_out)
            val_out = jnp.where(lane == t, m, val_out)
            cur = jnp.where(col == a, -jnp.inf, cur)
        e = jnp.where(lane < TOPK, jnp.exp(val_out - val_out[:, :1]), 0.0)
        s = jnp.sum(e, axis=-1, keepdims=True)
        idx_ref[...] = idx_out
        wgt_ref[...] = e / s


G = 8


def _attn_kernel(ridx_ref, q_ref, kv_ref, rw_ref, wo_ref, bo_ref, o_ref,
                 acc_ref):
    b = pl.program_id(0)
    jj = pl.program_id(1)
    lane = jax.lax.broadcasted_iota(jnp.int32, (1, 128), 1)
    for w in range(G):
        row = b * P2 + jj * G + w
        q = q_ref[w * W2:(w + 1) * W2, :]
        rww = rw_ref[w:w + 1, :]
        wts = [jnp.sum(jnp.where(lane == t, rww, 0.0)) for t in range(TOPK)]
        kvt = [kv_ref[0, pl.ds(ridx_ref[row, t] * W2, W2), :]
               for t in range(TOPK)]
        ls = [_dot_t(q, kvt[t][:, :QK]) * (wts[t] * SCALE)
              for t in range(TOPK)]
        l = jnp.concatenate(ls, axis=1)
        m = jnp.max(l, axis=-1, keepdims=True)
        p = jnp.exp(l - m)
        s = jnp.sum(p, axis=-1, keepdims=True)
        acc = _dot((p[:, :W2] * wts[0]).astype(jnp.bfloat16), kvt[0][:, QK:])
        for t in range(1, TOPK):
            pt = (p[:, t * W2:(t + 1) * W2] * wts[t]).astype(jnp.bfloat16)
            acc += _dot(pt, kvt[t][:, QK:])
        acc_ref[w * W2:(w + 1) * W2, :] = (acc / s).astype(jnp.bfloat16)
    o_ref[...] = _dot(acc_ref[...], wo_ref[...]) + bo_ref[...]


def kernel(x, W_qkv, b_qkv, W_o, b_o):
    n, p2, w2, dim = x.shape
    rows = n * p2 * w2
    x2 = x.reshape(rows, dim)
    b2 = b_qkv.reshape(1, 2 * QK + DIM)

    gb = P2 // PB
    q2, kv2, r_idx, r_wgt = pl.pallas_call(
        _qkv_kernel,
        grid=(rows // (PB * W2),),
        in_specs=[
            pl.BlockSpec((PB * W2, DIM), lambda g: (g, 0)),
            pl.BlockSpec((DIM, 2 * QK + DIM), lambda g: (0, 0)),
            pl.BlockSpec((1, 2 * QK + DIM), lambda g: (0, 0)),
        ],
        out_specs=[
            pl.BlockSpec((PB * W2, QK), lambda g: (g, 0)),
            pl.BlockSpec((PB * W2, KV), lambda g: (g, 0)),
            pl.BlockSpec((P2, 128), lambda g: (g // gb, 0)),
            pl.BlockSpec((P2, 128), lambda g: (g // gb, 0)),
        ],
        out_shape=[
            jax.ShapeDtypeStruct((rows, QK), jnp.bfloat16),
            jax.ShapeDtypeStruct((rows, KV), jnp.bfloat16),
            jax.ShapeDtypeStruct((n * p2, 128), jnp.int32),
            jax.ShapeDtypeStruct((n * p2, 128), jnp.float32),
        ],
        scratch_shapes=[
            pltpu.VMEM((P2, QK), jnp.float32),
            pltpu.VMEM((P2, QK), jnp.float32),
        ],
        compiler_params=pltpu.CompilerParams(
            dimension_semantics=("arbitrary",)),
    )(x2, W_qkv, b2)

    bo2 = b_o.reshape(1, DIM)

    out = pl.pallas_call(
        _attn_kernel,
        grid_spec=pltpu.PrefetchScalarGridSpec(
            num_scalar_prefetch=1,
            grid=(n, p2 // G),
            in_specs=[
                pl.BlockSpec((G * W2, QK),
                             lambda b, jj, ridx: (b * (P2 // G) + jj, 0)),
                pl.BlockSpec((1, P2 * W2, KV), lambda b, jj, ridx: (b, 0, 0)),
                pl.BlockSpec((G, 128),
                             lambda b, jj, ridx: (b * (P2 // G) + jj, 0)),
                pl.BlockSpec((DIM, DIM), lambda b, jj, ridx: (0, 0)),
                pl.BlockSpec((1, DIM), lambda b, jj, ridx: (0, 0)),
            ],
            out_specs=pl.BlockSpec(
                (G * W2, DIM), lambda b, jj, ridx: (b * (P2 // G) + jj, 0)),
            scratch_shapes=[pltpu.VMEM((G * W2, DIM), jnp.bfloat16)],
        ),
        out_shape=jax.ShapeDtypeStruct((rows, DIM), jnp.float32),
        compiler_params=pltpu.CompilerParams(
            dimension_semantics=("arbitrary", "arbitrary"),
            vmem_limit_bytes=100 * 1024 * 1024),
    )(r_idx, q2, kv2.reshape(n, p2 * w2, KV), r_wgt,
      W_o.astype(jnp.bfloat16), bo2)

    return out.reshape(n, p2, w2, dim)


# speedup vs baseline: 3.1928x; 1.0012x over previous
import jax
import jax.numpy as jnp
from jax.experimental import pallas as pl
from jax.experimental.pallas import tpu as pltpu

N = 2
P2 = 64
W2 = 64
DIM = 1024
QK = 1024
KV = 2048
TOPK = 4
SCALE = QK ** -0.5
PB = 8


def _dot(a, b, precision=None):
    return jax.lax.dot_general(a, b, (((1,), (0,)), ((), ())),
                               preferred_element_type=jnp.float32,
                               precision=precision)


def _dot_t(a, b):
    return jax.lax.dot_general(a, b, (((1,), (1,)), ((), ())),
                               preferred_element_type=jnp.float32)


def _qkv_kernel(x_ref, w_ref, b_ref, q_ref, kv_ref, idx_ref, wgt_ref,
                qw_s, kw_s):
    g = pl.program_id(0)
    gb = P2 // PB
    x = x_ref[...]
    qkv = _dot(x, w_ref[...]) + b_ref[...]
    q = qkv[:, :QK]
    kv = qkv[:, QK:]
    q_ref[...] = q.astype(jnp.bfloat16)
    kv_ref[...] = kv.astype(jnp.bfloat16)
    r = (g % gb) * PB
    qw_s[pl.ds(r, PB), :] = jnp.mean(q.reshape(PB, W2, QK), axis=1)
    kw_s[pl.ds(r, PB), :] = jnp.mean(kv[:, :QK].reshape(PB, W2, QK), axis=1)

    @pl.when(g % gb == gb - 1)
    def _():
        logit = _dot_t(qw_s[...] * SCALE, kw_s[...])
        col = jax.lax.broadcasted_iota(jnp.int32, (P2, P2), 1)
        lane = jax.lax.broadcasted_iota(jnp.int32, (P2, 128), 1)
        idx_out = jnp.zeros((P2, 128), jnp.int32)
        val_out = jnp.zeros((P2, 128), jnp.float32)
        cur = logit
        for t in range(TOPK):
            m = jnp.max(cur, axis=-1, keepdims=True)
            a = jnp.min(jnp.where(cur == m, col, P2), axis=-1, keepdims=True)
            idx_out = jnp.where(lane == t, a, idx_out)
            val_out = jnp.where(lane == t, m, val_out)
            cur = jnp.where(col == a, -jnp.inf, cur)
        e = jnp.where(lane < TOPK, jnp.exp(val_out - val_out[:, :1]), 0.0)
        s = jnp.sum(e, axis=-1, keepdims=True)
        idx_ref[...] = idx_out
        wgt_ref[...] = e / s


G = 8


def _attn_kernel(ridx_ref, q_ref, kv_ref, rw_ref, wo_ref, bo_ref, o_ref):
    b = pl.program_id(0)
    jj = pl.program_id(1)
    lane = jax.lax.broadcasted_iota(jnp.int32, (1, 128), 1)
    outs = []
    for w in range(G):
        row = b * P2 + jj * G + w
        q = q_ref[w * W2:(w + 1) * W2, :]
        rww = rw_ref[w:w + 1, :]
        wts = [jnp.sum(jnp.where(lane == t, rww, 0.0)) for t in range(TOPK)]
        kvt = [kv_ref[0, pl.ds(ridx_ref[row, t] * W2, W2), :]
               for t in range(TOPK)]
        ls = [_dot_t(q, kvt[t][:, :QK]) * (wts[t] * SCALE)
              for t in range(TOPK)]
        l = jnp.concatenate(ls, axis=1)
        m = jnp.max(l, axis=-1, keepdims=True)
        p = jnp.exp(l - m)
        s = jnp.sum(p, axis=-1, keepdims=True)
        acc = _dot((p[:, :W2] * wts[0]).astype(jnp.bfloat16), kvt[0][:, QK:])
        for t in range(1, TOPK):
            pt = (p[:, t * W2:(t + 1) * W2] * wts[t]).astype(jnp.bfloat16)
            acc += _dot(pt, kvt[t][:, QK:])
        outs.append((acc / s).astype(jnp.bfloat16))
    o_ref[...] = _dot(jnp.concatenate(outs, axis=0), wo_ref[...]) + bo_ref[...]


def kernel(x, W_qkv, b_qkv, W_o, b_o):
    n, p2, w2, dim = x.shape
    rows = n * p2 * w2
    x2 = x.reshape(rows, dim)
    b2 = b_qkv.reshape(1, 2 * QK + DIM)

    gb = P2 // PB
    q2, kv2, r_idx, r_wgt = pl.pallas_call(
        _qkv_kernel,
        grid=(rows // (PB * W2),),
        in_specs=[
            pl.BlockSpec((PB * W2, DIM), lambda g: (g, 0)),
            pl.BlockSpec((DIM, 2 * QK + DIM), lambda g: (0, 0)),
            pl.BlockSpec((1, 2 * QK + DIM), lambda g: (0, 0)),
        ],
        out_specs=[
            pl.BlockSpec((PB * W2, QK), lambda g: (g, 0)),
            pl.BlockSpec((PB * W2, KV), lambda g: (g, 0)),
            pl.BlockSpec((P2, 128), lambda g: (g // gb, 0)),
            pl.BlockSpec((P2, 128), lambda g: (g // gb, 0)),
        ],
        out_shape=[
            jax.ShapeDtypeStruct((rows, QK), jnp.bfloat16),
            jax.ShapeDtypeStruct((rows, KV), jnp.bfloat16),
            jax.ShapeDtypeStruct((n * p2, 128), jnp.int32),
            jax.ShapeDtypeStruct((n * p2, 128), jnp.float32),
        ],
        scratch_shapes=[
            pltpu.VMEM((P2, QK), jnp.float32),
            pltpu.VMEM((P2, QK), jnp.float32),
        ],
        compiler_params=pltpu.CompilerParams(
            dimension_semantics=("arbitrary",)),
    )(x2, W_qkv, b2)

    bo2 = b_o.reshape(1, DIM)

    out = pl.pallas_call(
        _attn_kernel,
        grid_spec=pltpu.PrefetchScalarGridSpec(
            num_scalar_prefetch=1,
            grid=(n, p2 // G),
            in_specs=[
                pl.BlockSpec((G * W2, QK),
                             lambda b, jj, ridx: (b * (P2 // G) + jj, 0)),
                pl.BlockSpec((1, P2 * W2, KV), lambda b, jj, ridx: (b, 0, 0)),
                pl.BlockSpec((G, 128),
                             lambda b, jj, ridx: (b * (P2 // G) + jj, 0)),
                pl.BlockSpec((DIM, DIM), lambda b, jj, ridx: (0, 0)),
                pl.BlockSpec((1, DIM), lambda b, jj, ridx: (0, 0)),
            ],
            out_specs=pl.BlockSpec(
                (G * W2, DIM), lambda b, jj, ridx: (b * (P2 // G) + jj, 0)),
        ),
        out_shape=jax.ShapeDtypeStruct((rows, DIM), jnp.float32),
        compiler_params=pltpu.CompilerParams(
            dimension_semantics=("arbitrary", "arbitrary"),
            vmem_limit_bytes=100 * 1024 * 1024),
    )(r_idx, q2, kv2.reshape(n, p2 * w2, KV), r_wgt,
      W_o.astype(jnp.bfloat16), bo2)

    return out.reshape(n, p2, w2, dim)


# speedup vs baseline: 4.1741x; 1.3073x over previous
import jax
import jax.numpy as jnp
from jax.experimental import pallas as pl
from jax.experimental.pallas import tpu as pltpu

N = 2
P2 = 64
W2 = 64
DIM = 1024
QK = 1024
KV = 2048
TOPK = 4
SCALE = QK ** -0.5
PB = 8


def _dot(a, b, precision=None):
    return jax.lax.dot_general(a, b, (((1,), (0,)), ((), ())),
                               preferred_element_type=jnp.float32,
                               precision=precision)


def _dot_t(a, b):
    return jax.lax.dot_general(a, b, (((1,), (1,)), ((), ())),
                               preferred_element_type=jnp.float32)


def _qkv_kernel(x_ref, w_ref, b_ref, q_ref, kv_ref, idx_ref, wgt_ref,
                qw_s, kw_s):
    g = pl.program_id(0)
    gb = P2 // PB
    x = x_ref[...]
    qkv = _dot(x, w_ref[...]) + b_ref[...]
    q = qkv[:, :QK]
    kv = qkv[:, QK:]
    q_ref[...] = q.astype(jnp.bfloat16)
    kv_ref[...] = kv.astype(jnp.bfloat16)
    r = (g % gb) * PB
    qw_s[pl.ds(r, PB), :] = jnp.mean(q.reshape(PB, W2, QK), axis=1)
    kw_s[pl.ds(r, PB), :] = jnp.mean(kv[:, :QK].reshape(PB, W2, QK), axis=1)

    @pl.when(g % gb == gb - 1)
    def _():
        logit = _dot_t(qw_s[...] * SCALE, kw_s[...])
        col = jax.lax.broadcasted_iota(jnp.int32, (P2, P2), 1)
        lane = jax.lax.broadcasted_iota(jnp.int32, (P2, 128), 1)
        idx_out = jnp.zeros((P2, 128), jnp.int32)
        val_out = jnp.zeros((P2, 128), jnp.float32)
        cur = logit
        for t in range(TOPK):
            m = jnp.max(cur, axis=-1, keepdims=True)
            a = jnp.min(jnp.where(cur == m, col, P2), axis=-1, keepdims=True)
            idx_out = jnp.where(lane == t, a, idx_out)
            val_out = jnp.where(lane == t, m, val_out)
            cur = jnp.where(col == a, -jnp.inf, cur)
        e = jnp.where(lane < TOPK, jnp.exp(val_out - val_out[:, :1]), 0.0)
        s = jnp.sum(e, axis=-1, keepdims=True)
        idx_ref[...] = idx_out
        wgt_ref[...] = e / s


G = 8


def _attn_kernel(ridx_ref, q_ref, kv_ref, rw_ref, wo_ref, bo_ref, o_ref):
    b = pl.program_id(0)
    jj = pl.program_id(1)
    lane = jax.lax.broadcasted_iota(jnp.int32, (1, 128), 1)
    wts_all, kvt_all, l_all = [], [], []
    for w in range(G):
        row = b * P2 + jj * G + w
        q = q_ref[w * W2:(w + 1) * W2, :]
        rww = rw_ref[w:w + 1, :]
        wts = [jnp.sum(jnp.where(lane == t, rww, 0.0)) for t in range(TOPK)]
        kvt = [kv_ref[0, pl.ds(ridx_ref[row, t] * W2, W2), :]
               for t in range(TOPK)]
        ls = [_dot_t(q, kvt[t][:, :QK]) * (wts[t] * SCALE)
              for t in range(TOPK)]
        wts_all.append(wts)
        kvt_all.append(kvt)
        l_all.append(jnp.concatenate(ls, axis=1))
    p_all = []
    for w in range(G):
        l = l_all[w]
        m = jnp.max(l, axis=-1, keepdims=True)
        p = jnp.exp(l - m)
        s = jnp.sum(p, axis=-1, keepdims=True)
        p_all.append((p, s))
    outs = []
    for w in range(G):
        p, s = p_all[w]
        wts, kvt = wts_all[w], kvt_all[w]
        acc = _dot((p[:, :W2] * wts[0]).astype(jnp.bfloat16), kvt[0][:, QK:])
        for t in range(1, TOPK):
            pt = (p[:, t * W2:(t + 1) * W2] * wts[t]).astype(jnp.bfloat16)
            acc += _dot(pt, kvt[t][:, QK:])
        outs.append((acc / s).astype(jnp.bfloat16))
    o_ref[...] = _dot(jnp.concatenate(outs, axis=0), wo_ref[...]) + bo_ref[...]


def kernel(x, W_qkv, b_qkv, W_o, b_o):
    n, p2, w2, dim = x.shape
    rows = n * p2 * w2
    x2 = x.reshape(rows, dim)
    b2 = b_qkv.reshape(1, 2 * QK + DIM)

    gb = P2 // PB
    q2, kv2, r_idx, r_wgt = pl.pallas_call(
        _qkv_kernel,
        grid=(rows // (PB * W2),),
        in_specs=[
            pl.BlockSpec((PB * W2, DIM), lambda g: (g, 0)),
            pl.BlockSpec((DIM, 2 * QK + DIM), lambda g: (0, 0)),
            pl.BlockSpec((1, 2 * QK + DIM), lambda g: (0, 0)),
        ],
        out_specs=[
            pl.BlockSpec((PB * W2, QK), lambda g: (g, 0)),
            pl.BlockSpec((PB * W2, KV), lambda g: (g, 0)),
            pl.BlockSpec((P2, 128), lambda g: (g // gb, 0)),
            pl.BlockSpec((P2, 128), lambda g: (g // gb, 0)),
        ],
        out_shape=[
            jax.ShapeDtypeStruct((rows, QK), jnp.bfloat16),
            jax.ShapeDtypeStruct((rows, KV), jnp.bfloat16),
            jax.ShapeDtypeStruct((n * p2, 128), jnp.int32),
            jax.ShapeDtypeStruct((n * p2, 128), jnp.float32),
        ],
        scratch_shapes=[
            pltpu.VMEM((P2, QK), jnp.float32),
            pltpu.VMEM((P2, QK), jnp.float32),
        ],
        compiler_params=pltpu.CompilerParams(
            dimension_semantics=("arbitrary",)),
    )(x2, W_qkv, b2)

    bo2 = b_o.reshape(1, DIM)

    out = pl.pallas_call(
        _attn_kernel,
        grid_spec=pltpu.PrefetchScalarGridSpec(
            num_scalar_prefetch=1,
            grid=(n, p2 // G),
            in_specs=[
                pl.BlockSpec((G * W2, QK),
                             lambda b, jj, ridx: (b * (P2 // G) + jj, 0)),
                pl.BlockSpec((1, P2 * W2, KV), lambda b, jj, ridx: (b, 0, 0)),
                pl.BlockSpec((G, 128),
                             lambda b, jj, ridx: (b * (P2 // G) + jj, 0)),
                pl.BlockSpec((DIM, DIM), lambda b, jj, ridx: (0, 0)),
                pl.BlockSpec((1, DIM), lambda b, jj, ridx: (0, 0)),
            ],
            out_specs=pl.BlockSpec(
                (G * W2, DIM), lambda b, jj, ridx: (b * (P2 // G) + jj, 0)),
        ),
        out_shape=jax.ShapeDtypeStruct((rows, DIM), jnp.float32),
        compiler_params=pltpu.CompilerParams(
            dimension_semantics=("parallel", "parallel"),
            vmem_limit_bytes=100 * 1024 * 1024),
    )(r_idx, q2, kv2.reshape(n, p2 * w2, KV), r_wgt,
      W_o.astype(jnp.bfloat16), bo2)

    return out.reshape(n, p2, w2, dim)


# speedup vs baseline: 4.3399x; 1.0397x over previous
import jax
import jax.numpy as jnp
from jax.experimental import pallas as pl
from jax.experimental.pallas import tpu as pltpu

N = 2
P2 = 64
W2 = 64
DIM = 1024
QK = 1024
KV = 2048
TOPK = 4
SCALE = QK ** -0.5
PB = 8


def _dot(a, b, precision=None):
    return jax.lax.dot_general(a, b, (((1,), (0,)), ((), ())),
                               preferred_element_type=jnp.float32,
                               precision=precision)


def _dot_t(a, b):
    return jax.lax.dot_general(a, b, (((1,), (1,)), ((), ())),
                               preferred_element_type=jnp.float32)


def _qkv_kernel(x_ref, w_ref, b_ref, q_ref, kv_ref, idx_ref, wgt_ref,
                qw_s, kw_s):
    g = pl.program_id(0)
    gb = P2 // PB
    x = x_ref[...]
    qkv = _dot(x, w_ref[...]) + b_ref[...]
    q = qkv[:, :QK]
    kv = qkv[:, QK:]
    q_ref[...] = q.astype(jnp.bfloat16)
    kv_ref[...] = kv.astype(jnp.bfloat16)
    r = (g % gb) * PB
    qw_s[pl.ds(r, PB), :] = jnp.mean(q.reshape(PB, W2, QK), axis=1)
    kw_s[pl.ds(r, PB), :] = jnp.mean(kv[:, :QK].reshape(PB, W2, QK), axis=1)

    @pl.when(g % gb == gb - 1)
    def _():
        logit = _dot_t(qw_s[...] * SCALE, kw_s[...])
        col = jax.lax.broadcasted_iota(jnp.int32, (P2, P2), 1)
        lane = jax.lax.broadcasted_iota(jnp.int32, (P2, 128), 1)
        idx_out = jnp.zeros((P2, 128), jnp.int32)
        val_out = jnp.zeros((P2, 128), jnp.float32)
        cur = logit
        for t in range(TOPK):
            m = jnp.max(cur, axis=-1, keepdims=True)
            a = jnp.min(jnp.where(cur == m, col, P2), axis=-1, keepdims=True)
            idx_out = jnp.where(lane == t, a, idx_out)
            val_out = jnp.where(lane == t, m, val_out)
            cur = jnp.where(col == a, -jnp.inf, cur)
        e = jnp.where(lane < TOPK, jnp.exp(val_out - val_out[:, :1]), 0.0)
        s = jnp.sum(e, axis=-1, keepdims=True)
        idx_ref[...] = idx_out
        wgt_ref[...] = e / s


G = 16


def _attn_kernel(ridx_ref, q_ref, kv_ref, rw_ref, wo_ref, bo_ref, o_ref):
    b = pl.program_id(0)
    jj = pl.program_id(1)
    lane = jax.lax.broadcasted_iota(jnp.int32, (1, 128), 1)
    wts_all, kvt_all, l_all = [], [], []
    for w in range(G):
        row = b * P2 + jj * G + w
        q = q_ref[w * W2:(w + 1) * W2, :]
        rww = rw_ref[w:w + 1, :]
        wts = [jnp.sum(jnp.where(lane == t, rww, 0.0)) for t in range(TOPK)]
        kvt = [kv_ref[0, pl.ds(ridx_ref[row, t] * W2, W2), :]
               for t in range(TOPK)]
        ls = [_dot_t(q, kvt[t][:, :QK]) * (wts[t] * SCALE)
              for t in range(TOPK)]
        wts_all.append(wts)
        kvt_all.append(kvt)
        l_all.append(jnp.concatenate(ls, axis=1))
    p_all = []
    for w in range(G):
        l = l_all[w]
        m = jnp.max(l, axis=-1, keepdims=True)
        p = jnp.exp(l - m)
        s = jnp.sum(p, axis=-1, keepdims=True)
        p_all.append((p, s))
    outs = []
    for w in range(G):
        p, s = p_all[w]
        wts, kvt = wts_all[w], kvt_all[w]
        acc = _dot((p[:, :W2] * wts[0]).astype(jnp.bfloat16), kvt[0][:, QK:])
        for t in range(1, TOPK):
            pt = (p[:, t * W2:(t + 1) * W2] * wts[t]).astype(jnp.bfloat16)
            acc += _dot(pt, kvt[t][:, QK:])
        outs.append((acc / s).astype(jnp.bfloat16))
    o_ref[...] = _dot(jnp.concatenate(outs, axis=0), wo_ref[...]) + bo_ref[...]


def kernel(x, W_qkv, b_qkv, W_o, b_o):
    n, p2, w2, dim = x.shape
    rows = n * p2 * w2
    x2 = x.reshape(rows, dim)
    b2 = b_qkv.reshape(1, 2 * QK + DIM)

    gb = P2 // PB
    q2, kv2, r_idx, r_wgt = pl.pallas_call(
        _qkv_kernel,
        grid=(rows // (PB * W2),),
        in_specs=[
            pl.BlockSpec((PB * W2, DIM), lambda g: (g, 0)),
            pl.BlockSpec((DIM, 2 * QK + DIM), lambda g: (0, 0)),
            pl.BlockSpec((1, 2 * QK + DIM), lambda g: (0, 0)),
        ],
        out_specs=[
            pl.BlockSpec((PB * W2, QK), lambda g: (g, 0)),
            pl.BlockSpec((PB * W2, KV), lambda g: (g, 0)),
            pl.BlockSpec((P2, 128), lambda g: (g // gb, 0)),
            pl.BlockSpec((P2, 128), lambda g: (g // gb, 0)),
        ],
        out_shape=[
            jax.ShapeDtypeStruct((rows, QK), jnp.bfloat16),
            jax.ShapeDtypeStruct((rows, KV), jnp.bfloat16),
            jax.ShapeDtypeStruct((n * p2, 128), jnp.int32),
            jax.ShapeDtypeStruct((n * p2, 128), jnp.float32),
        ],
        scratch_shapes=[
            pltpu.VMEM((P2, QK), jnp.float32),
            pltpu.VMEM((P2, QK), jnp.float32),
        ],
        compiler_params=pltpu.CompilerParams(
            dimension_semantics=("arbitrary",)),
    )(x2, W_qkv, b2)

    bo2 = b_o.reshape(1, DIM)

    out = pl.pallas_call(
        _attn_kernel,
        grid_spec=pltpu.PrefetchScalarGridSpec(
            num_scalar_prefetch=1,
            grid=(n, p2 // G),
            in_specs=[
                pl.BlockSpec((G * W2, QK),
                             lambda b, jj, ridx: (b * (P2 // G) + jj, 0)),
                pl.BlockSpec((1, P2 * W2, KV), lambda b, jj, ridx: (b, 0, 0)),
                pl.BlockSpec((G, 128),
                             lambda b, jj, ridx: (b * (P2 // G) + jj, 0)),
                pl.BlockSpec((DIM, DIM), lambda b, jj, ridx: (0, 0)),
                pl.BlockSpec((1, DIM), lambda b, jj, ridx: (0, 0)),
            ],
            out_specs=pl.BlockSpec(
                (G * W2, DIM), lambda b, jj, ridx: (b * (P2 // G) + jj, 0)),
        ),
        out_shape=jax.ShapeDtypeStruct((rows, DIM), jnp.float32),
        compiler_params=pltpu.CompilerParams(
            dimension_semantics=("parallel", "parallel"),
            vmem_limit_bytes=100 * 1024 * 1024),
    )(r_idx, q2, kv2.reshape(n, p2 * w2, KV), r_wgt,
      W_o.astype(jnp.bfloat16), bo2)

    return out.reshape(n, p2, w2, dim)


# speedup vs baseline: 4.3698x; 1.0069x over previous
import jax
import jax.numpy as jnp
from jax.experimental import pallas as pl
from jax.experimental.pallas import tpu as pltpu

N = 2
P2 = 64
W2 = 64
DIM = 1024
QK = 1024
KV = 2048
TOPK = 4
SCALE = QK ** -0.5
PB = 16


def _dot(a, b, precision=None):
    return jax.lax.dot_general(a, b, (((1,), (0,)), ((), ())),
                               preferred_element_type=jnp.float32,
                               precision=precision)


def _dot_t(a, b):
    return jax.lax.dot_general(a, b, (((1,), (1,)), ((), ())),
                               preferred_element_type=jnp.float32)


def _qkv_kernel(x_ref, w_ref, b_ref, q_ref, kv_ref, idx_ref, wgt_ref,
                qw_s, kw_s):
    g = pl.program_id(0)
    gb = P2 // PB
    x = x_ref[...]
    qkv = _dot(x, w_ref[...]) + b_ref[...]
    q = qkv[:, :QK]
    kv = qkv[:, QK:]
    q_ref[...] = q.astype(jnp.bfloat16)
    kv_ref[...] = kv.astype(jnp.bfloat16)
    r = (g % gb) * PB
    qw_s[pl.ds(r, PB), :] = jnp.mean(q.reshape(PB, W2, QK), axis=1)
    kw_s[pl.ds(r, PB), :] = jnp.mean(kv[:, :QK].reshape(PB, W2, QK), axis=1)

    @pl.when(g % gb == gb - 1)
    def _():
        logit = _dot_t(qw_s[...] * SCALE, kw_s[...])
        col = jax.lax.broadcasted_iota(jnp.int32, (P2, P2), 1)
        lane = jax.lax.broadcasted_iota(jnp.int32, (P2, 128), 1)
        idx_out = jnp.zeros((P2, 128), jnp.int32)
        val_out = jnp.zeros((P2, 128), jnp.float32)
        cur = logit
        for t in range(TOPK):
            m = jnp.max(cur, axis=-1, keepdims=True)
            a = jnp.min(jnp.where(cur == m, col, P2), axis=-1, keepdims=True)
            idx_out = jnp.where(lane == t, a, idx_out)
            val_out = jnp.where(lane == t, m, val_out)
            cur = jnp.where(col == a, -jnp.inf, cur)
        e = jnp.where(lane < TOPK, jnp.exp(val_out - val_out[:, :1]), 0.0)
        s = jnp.sum(e, axis=-1, keepdims=True)
        idx_ref[...] = idx_out
        wgt_ref[...] = e / s


G = 16


def _attn_kernel(ridx_ref, q_ref, kv_ref, rw_ref, wo_ref, bo_ref, o_ref):
    b = pl.program_id(0)
    jj = pl.program_id(1)
    lane = jax.lax.broadcasted_iota(jnp.int32, (1, 128), 1)
    wts_all, kvt_all, l_all = [], [], []
    for w in range(G):
        row = b * P2 + jj * G + w
        q = q_ref[w * W2:(w + 1) * W2, :]
        rww = rw_ref[w:w + 1, :]
        wts = [jnp.sum(jnp.where(lane == t, rww, 0.0)) for t in range(TOPK)]
        kvt = [kv_ref[0, pl.ds(ridx_ref[row, t] * W2, W2), :]
               for t in range(TOPK)]
        ls = [_dot_t(q, kvt[t][:, :QK]) * (wts[t] * SCALE)
              for t in range(TOPK)]
        wts_all.append(wts)
        kvt_all.append(kvt)
        l_all.append(jnp.concatenate(ls, axis=1))
    p_all = []
    for w in range(G):
        l = l_all[w]
        m = jnp.max(l, axis=-1, keepdims=True)
        p = jnp.exp(l - m)
        s = jnp.sum(p, axis=-1, keepdims=True)
        p_all.append((p, s))
    outs = []
    for w in range(G):
        p, s = p_all[w]
        wts, kvt = wts_all[w], kvt_all[w]
        acc = _dot((p[:, :W2] * wts[0]).astype(jnp.bfloat16), kvt[0][:, QK:])
        for t in range(1, TOPK):
            pt = (p[:, t * W2:(t + 1) * W2] * wts[t]).astype(jnp.bfloat16)
            acc += _dot(pt, kvt[t][:, QK:])
        outs.append((acc / s).astype(jnp.bfloat16))
    o_ref[...] = _dot(jnp.concatenate(outs, axis=0), wo_ref[...]) + bo_ref[...]


def kernel(x, W_qkv, b_qkv, W_o, b_o):
    n, p2, w2, dim = x.shape
    rows = n * p2 * w2
    x2 = x.reshape(rows, dim)
    b2 = b_qkv.reshape(1, 2 * QK + DIM)

    gb = P2 // PB
    q2, kv2, r_idx, r_wgt = pl.pallas_call(
        _qkv_kernel,
        grid=(rows // (PB * W2),),
        in_specs=[
            pl.BlockSpec((PB * W2, DIM), lambda g: (g, 0)),
            pl.BlockSpec((DIM, 2 * QK + DIM), lambda g: (0, 0)),
            pl.BlockSpec((1, 2 * QK + DIM), lambda g: (0, 0)),
        ],
        out_specs=[
            pl.BlockSpec((PB * W2, QK), lambda g: (g, 0)),
            pl.BlockSpec((PB * W2, KV), lambda g: (g, 0)),
            pl.BlockSpec((P2, 128), lambda g: (g // gb, 0)),
            pl.BlockSpec((P2, 128), lambda g: (g // gb, 0)),
        ],
        out_shape=[
            jax.ShapeDtypeStruct((rows, QK), jnp.bfloat16),
            jax.ShapeDtypeStruct((rows, KV), jnp.bfloat16),
            jax.ShapeDtypeStruct((n * p2, 128), jnp.int32),
            jax.ShapeDtypeStruct((n * p2, 128), jnp.float32),
        ],
        scratch_shapes=[
            pltpu.VMEM((P2, QK), jnp.float32),
            pltpu.VMEM((P2, QK), jnp.float32),
        ],
        compiler_params=pltpu.CompilerParams(
            dimension_semantics=("arbitrary",)),
    )(x2, W_qkv, b2)

    bo2 = b_o.reshape(1, DIM)

    out = pl.pallas_call(
        _attn_kernel,
        grid_spec=pltpu.PrefetchScalarGridSpec(
            num_scalar_prefetch=1,
            grid=(n, p2 // G),
            in_specs=[
                pl.BlockSpec((G * W2, QK),
                             lambda b, jj, ridx: (b * (P2 // G) + jj, 0)),
                pl.BlockSpec((1, P2 * W2, KV), lambda b, jj, ridx: (b, 0, 0)),
                pl.BlockSpec((G, 128),
                             lambda b, jj, ridx: (b * (P2 // G) + jj, 0)),
                pl.BlockSpec((DIM, DIM), lambda b, jj, ridx: (0, 0)),
                pl.BlockSpec((1, DIM), lambda b, jj, ridx: (0, 0)),
            ],
            out_specs=pl.BlockSpec(
                (G * W2, DIM), lambda b, jj, ridx: (b * (P2 // G) + jj, 0)),
        ),
        out_shape=jax.ShapeDtypeStruct((rows, DIM), jnp.float32),
        compiler_params=pltpu.CompilerParams(
            dimension_semantics=("parallel", "parallel"),
            vmem_limit_bytes=100 * 1024 * 1024),
    )(r_idx, q2, kv2.reshape(n, p2 * w2, KV), r_wgt,
      W_o.astype(jnp.bfloat16), bo2)

    return out.reshape(n, p2, w2, dim)
